# Initial kernel scaffold; baseline (speedup 1.0000x reference)
import jax
import jax.numpy as jnp
from jax.experimental import pallas as pl


def kernel(feats, batch_idx, coords):
    raise NotImplementedError("write your pallas kernel here")


# speedup vs baseline: 2.8762x; 2.8762x over previous
import functools

import jax
import jax.numpy as jnp
from jax import lax
from jax.experimental import pallas as pl
from jax.experimental.pallas import tpu as pltpu
from jax.experimental.pallas import tpu_sc as plsc

_B, _C, _H, _W = 4, 64, 256, 256
_N = 262144
_M = _B * _H * _W
_NC, _NS = 2, 16
_NW = _NC * _NS
_CPT = _M // _NW
_CHUNK = 2048
_NCHUNKS = _N // _CHUNK
_GC = 128
_NG = _CPT // _GC


def _sc_body(cell_hbm, feats_hbm, nhwc_out, mask_out,
             cellbuf, winner, idxbuf, rowsbuf, maskbuf,
             sem_in, sem_g, sem_rows, sem_mask):
    wid = lax.axis_index("s") * _NC + lax.axis_index("c")
    cell_base = wid * _CPT
    iota = lax.iota(jnp.int32, 16)

    def init_body(i, carry):
        for u in range(4):
            off = i * 64 + u * 16
            winner[pl.ds(off, 16)] = jnp.bitwise_not(cell_base + off + iota)
        return carry
    lax.fori_loop(0, _CPT // 64, init_body, 0)

    pltpu.async_copy(cell_hbm.at[pl.ds(0, _CHUNK)],
                     cellbuf.at[pl.ds(0, _CHUNK)], sem_in)

    def chunk_body(c, carry):
        slot = lax.rem(c, 2)

        @pl.when(c + 1 < _NCHUNKS)
        def _():
            nslot = lax.rem(c + 1, 2)
            pltpu.async_copy(cell_hbm.at[pl.ds((c + 1) * _CHUNK, _CHUNK)],
                             cellbuf.at[pl.ds(nslot * _CHUNK, _CHUNK)],
                             sem_in)

        pltpu.make_async_copy(cell_hbm.at[pl.ds(c * _CHUNK, _CHUNK)],
                              cellbuf.at[pl.ds(slot * _CHUNK, _CHUNK)],
                              sem_in).wait()

        def vec_body(v, vcarry):
            for u in range(4):
                o = v * 64 + u * 16
                cells = cellbuf[pl.ds(slot * _CHUNK + o, 16)]
                local = cells - cell_base
                m = (local >= 0) & (local < _CPT)
                lc = jnp.bitwise_and(local, _CPT - 1)
                ivec = c * _CHUNK + o + iota
                plsc.store_scatter(winner, [lc], ivec, mask=m)
                g = plsc.load_gather(winner, [lc], mask=m)
                m2 = m & (g < ivec)
                plsc.store_scatter(winner, [lc], ivec, mask=m2)
                g2 = plsc.load_gather(winner, [lc], mask=m2)
                m3 = m2 & (g2 < ivec)
                plsc.store_scatter(winner, [lc], ivec, mask=m3)
            return vcarry
        lax.fori_loop(0, _CHUNK // 64, vec_body, 0)
        return carry
    lax.fori_loop(0, _NCHUNKS, chunk_body, 0)

    def prep(gidx, slot):
        ib = idxbuf.at[slot]
        mb = maskbuf.at[slot]

        def pbody(v, carry):
            w = winner[pl.ds(gidx * _GC + v * 16, 16)]
            neg = w < 0
            ib[pl.ds(v * 16, 16)] = jnp.where(neg, jnp.bitwise_not(w), w)
            mb[pl.ds(v * 16, 16)] = jnp.where(
                neg, jnp.float32(0.0), jnp.float32(1.0))
            return carry
        lax.fori_loop(0, _GC // 16, pbody, 0)

    def gloop(g, carry):
        slot = lax.rem(g, 2)

        @pl.when(g >= 2)
        def _():
            rb = cell_base + (g - 2) * _GC
            pltpu.make_async_copy(rowsbuf.at[slot],
                                  nhwc_out.at[pl.ds(rb, _GC)],
                                  sem_rows).wait()
            pltpu.make_async_copy(maskbuf.at[slot],
                                  mask_out.at[pl.ds(rb, _GC)],
                                  sem_mask).wait()

        @pl.when(g < _NG)
        def _():
            prep(g, slot)
            pltpu.async_copy(feats_hbm.at[idxbuf.at[slot]],
                             rowsbuf.at[slot], sem_g)

        @pl.when(g >= 1)
        def _():
            pslot = lax.rem(g - 1, 2)
            pb = cell_base + (g - 1) * _GC
            pltpu.make_async_copy(feats_hbm.at[idxbuf.at[pslot]],
                                  rowsbuf.at[pslot], sem_g).wait()
            pltpu.async_copy(rowsbuf.at[pslot],
                             nhwc_out.at[pl.ds(pb, _GC)], sem_rows)
            pltpu.async_copy(maskbuf.at[pslot],
                             mask_out.at[pl.ds(pb, _GC)], sem_mask)
        return carry
    lax.fori_loop(0, _NG + 1, gloop, 0)

    lb = cell_base + (_NG - 1) * _GC
    lslot = (_NG - 1) % 2
    pltpu.make_async_copy(rowsbuf.at[lslot],
                          nhwc_out.at[pl.ds(lb, _GC)], sem_rows).wait()
    pltpu.make_async_copy(maskbuf.at[lslot],
                          mask_out.at[pl.ds(lb, _GC)], sem_mask).wait()


_sc_call = pl.kernel(
    _sc_body,
    mesh=plsc.VectorSubcoreMesh(core_axis_name="c", subcore_axis_name="s"),
    compiler_params=pltpu.CompilerParams(
        needs_layout_passes=False, use_tc_tiling_on_sc=False),
    out_type=(
        jax.ShapeDtypeStruct((_M, _C), jnp.float32),
        jax.ShapeDtypeStruct((_M,), jnp.float32),
    ),
    scratch_types=[
        pltpu.VMEM((2 * _CHUNK,), jnp.int32),
        pltpu.VMEM((_CPT,), jnp.int32),
        pltpu.VMEM((2, _GC), jnp.int32),
        pltpu.VMEM((2, _GC, _C), jnp.float32),
        pltpu.VMEM((2, _GC), jnp.float32),
        pltpu.SemaphoreType.DMA,
        pltpu.SemaphoreType.DMA,
        pltpu.SemaphoreType.DMA,
        pltpu.SemaphoreType.DMA,
    ],
)


def _tc_body(x_ref, m_ref, o_ref):
    x = x_ref[0]
    t = jnp.transpose(x)
    o_ref[0] = t * m_ref[0]


@functools.partial(jax.jit, donate_argnums=())
def _tc_call(nhwc3, mask2):
    return pl.pallas_call(
        _tc_body,
        grid=(_B, (_H * _W) // 2048),
        in_specs=[
            pl.BlockSpec((1, 2048, _C), lambda b, h: (b, h, 0)),
            pl.BlockSpec((1, 1, 2048), lambda b, h: (b, 0, h)),
        ],
        out_specs=pl.BlockSpec((1, _C, 2048), lambda b, h: (b, 0, h)),
        out_shape=jax.ShapeDtypeStruct((_B, _C, _H * _W), jnp.float32),
    )(nhwc3, mask2)


def kernel(feats, batch_idx, coords):
    cell = (batch_idx * (_H * _W)
            + coords[:, 0] * _W + coords[:, 1]).astype(jnp.int32)
    nhwc, maskv = _sc_call(cell, feats)
    out = _tc_call(nhwc.reshape(_B, _H * _W, _C),
                   maskv.reshape(_B, 1, _H * _W))
    return out.reshape(_B, _C, _H, _W)


# speedup vs baseline: 4.0115x; 1.3947x over previous
import functools

import jax
import jax.numpy as jnp
from jax import lax
from jax.experimental import pallas as pl
from jax.experimental.pallas import tpu as pltpu
from jax.experimental.pallas import tpu_sc as plsc

_B, _C, _H, _W = 4, 64, 256, 256
_N = 262144
_M = _B * _H * _W
_NC, _NS = 2, 16
_NW = _NC * _NS
_CPT = _M // _NW
_CHUNK = 2048
_NCHUNKS = _N // _CHUNK
_GC = 128
_NG = _CPT // _GC


def _sc_body(cell_hbm, feats_hbm, nhwc_out, mask_out,
             cellbuf, winner, idxbuf, rowsbuf, maskbuf,
             sem_in, sem_g, sem_rows, sem_mask):
    wid = lax.axis_index("s") * _NC + lax.axis_index("c")
    cell_base = wid * _CPT
    iota = lax.iota(jnp.int32, 16)

    def init_body(i, carry):
        for u in range(4):
            off = i * 64 + u * 16
            winner[pl.ds(off, 16)] = jnp.bitwise_not(cell_base + off + iota)
        return carry
    lax.fori_loop(0, _CPT // 64, init_body, 0)

    pltpu.async_copy(cell_hbm.at[pl.ds(0, _CHUNK)],
                     cellbuf.at[pl.ds(0, _CHUNK)], sem_in)

    def chunk_body(c, carry):
        slot = lax.rem(c, 2)

        @pl.when(c + 1 < _NCHUNKS)
        def _():
            nslot = lax.rem(c + 1, 2)
            pltpu.async_copy(cell_hbm.at[pl.ds((c + 1) * _CHUNK, _CHUNK)],
                             cellbuf.at[pl.ds(nslot * _CHUNK, _CHUNK)],
                             sem_in)

        pltpu.make_async_copy(cell_hbm.at[pl.ds(c * _CHUNK, _CHUNK)],
                              cellbuf.at[pl.ds(slot * _CHUNK, _CHUNK)],
                              sem_in).wait()

        def vec_body(v, vcarry):
            for u in range(4):
                o = v * 64 + u * 16
                cells = cellbuf[pl.ds(slot * _CHUNK + o, 16)]
                local = cells - cell_base
                m = (local >= 0) & (local < _CPT)
                lc = jnp.bitwise_and(local, _CPT - 1)
                ivec = c * _CHUNK + o + iota
                plsc.store_scatter(winner, [lc], ivec, mask=m)
            return vcarry
        lax.fori_loop(0, _CHUNK // 64, vec_body, 0)
        return carry
    lax.fori_loop(0, _NCHUNKS, chunk_body, 0)

    def prep(gidx, slot):
        ib = idxbuf.at[slot]
        mb = maskbuf.at[slot]

        def pbody(v, carry):
            w = winner[pl.ds(gidx * _GC + v * 16, 16)]
            neg = w < 0
            ib[pl.ds(v * 16, 16)] = jnp.where(neg, jnp.bitwise_not(w), w)
            mb[pl.ds(v * 16, 16)] = jnp.where(
                neg, jnp.float32(0.0), jnp.float32(1.0))
            return carry
        lax.fori_loop(0, _GC // 16, pbody, 0)

    def gloop(g, carry):
        slot = lax.rem(g, 2)

        @pl.when(g >= 2)
        def _():
            rb = cell_base + (g - 2) * _GC
            pltpu.make_async_copy(rowsbuf.at[slot],
                                  nhwc_out.at[pl.ds(rb, _GC)],
                                  sem_rows).wait()
            pltpu.make_async_copy(maskbuf.at[slot],
                                  mask_out.at[pl.ds(rb, _GC)],
                                  sem_mask).wait()

        @pl.when(g < _NG)
        def _():
            prep(g, slot)
            pltpu.async_copy(feats_hbm.at[idxbuf.at[slot]],
                             rowsbuf.at[slot], sem_g)

        @pl.when(g >= 1)
        def _():
            pslot = lax.rem(g - 1, 2)
            pb = cell_base + (g - 1) * _GC
            pltpu.make_async_copy(feats_hbm.at[idxbuf.at[pslot]],
                                  rowsbuf.at[pslot], sem_g).wait()
            pltpu.async_copy(rowsbuf.at[pslot],
                             nhwc_out.at[pl.ds(pb, _GC)], sem_rows)
            pltpu.async_copy(maskbuf.at[pslot],
                             mask_out.at[pl.ds(pb, _GC)], sem_mask)
        return carry
    lax.fori_loop(0, _NG + 1, gloop, 0)

    lb = cell_base + (_NG - 1) * _GC
    lslot = (_NG - 1) % 2
    pltpu.make_async_copy(rowsbuf.at[lslot],
                          nhwc_out.at[pl.ds(lb, _GC)], sem_rows).wait()
    pltpu.make_async_copy(maskbuf.at[lslot],
                          mask_out.at[pl.ds(lb, _GC)], sem_mask).wait()


_sc_call = pl.kernel(
    _sc_body,
    mesh=plsc.VectorSubcoreMesh(core_axis_name="c", subcore_axis_name="s"),
    compiler_params=pltpu.CompilerParams(
        needs_layout_passes=False, use_tc_tiling_on_sc=False),
    out_type=(
        jax.ShapeDtypeStruct((_M, _C), jnp.float32),
        jax.ShapeDtypeStruct((_M,), jnp.float32),
    ),
    scratch_types=[
        pltpu.VMEM((2 * _CHUNK,), jnp.int32),
        pltpu.VMEM((_CPT,), jnp.int32),
        pltpu.VMEM((2, _GC), jnp.int32),
        pltpu.VMEM((2, _GC, _C), jnp.float32),
        pltpu.VMEM((2, _GC), jnp.float32),
        pltpu.SemaphoreType.DMA,
        pltpu.SemaphoreType.DMA,
        pltpu.SemaphoreType.DMA,
        pltpu.SemaphoreType.DMA,
    ],
)


def _tc_body(x_ref, m_ref, o_ref):
    x = x_ref[0]
    t = jnp.transpose(x)
    o_ref[0] = t * m_ref[0]


@functools.partial(jax.jit, donate_argnums=())
def _tc_call(nhwc3, mask2):
    return pl.pallas_call(
        _tc_body,
        grid=(_B, (_H * _W) // 2048),
        in_specs=[
            pl.BlockSpec((1, 2048, _C), lambda b, h: (b, h, 0)),
            pl.BlockSpec((1, 1, 2048), lambda b, h: (b, 0, h)),
        ],
        out_specs=pl.BlockSpec((1, _C, 2048), lambda b, h: (b, 0, h)),
        out_shape=jax.ShapeDtypeStruct((_B, _C, _H * _W), jnp.float32),
    )(nhwc3, mask2)


def kernel(feats, batch_idx, coords):
    cell = (batch_idx * (_H * _W)
            + coords[:, 0] * _W + coords[:, 1]).astype(jnp.int32)
    nhwc, maskv = _sc_call(cell, feats)
    out = _tc_call(nhwc.reshape(_B, _H * _W, _C),
                   maskv.reshape(_B, 1, _H * _W))
    return out.reshape(_B, _C, _H, _W)


# speedup vs baseline: 6.3056x; 1.5719x over previous
import functools

import jax
import jax.numpy as jnp
from jax import lax
from jax.experimental import pallas as pl
from jax.experimental.pallas import tpu as pltpu
from jax.experimental.pallas import tpu_sc as plsc

_B, _C, _H, _W = 4, 64, 256, 256
_N = 262144
_M = _B * _H * _W
_NC, _NS = 2, 16
_NW = _NC * _NS
_CPT = _M // _NW
_CHUNK = 2048
_NCHUNKS = _N // _CHUNK
_GC = 128
_NG = _CPT // _GC


def _sc_body(cell_hbm, feats_hbm, table_out, mask_out,
             cellbuf, winner, idxbuf, rowsbuf, maskvm,
             sem_in, sem_g, sem_rows, sem_mask):
    wid = lax.axis_index("s") * _NC + lax.axis_index("c")
    cell_base = wid * _CPT
    iota = lax.iota(jnp.int32, 16)

    def init_body(i, carry):
        for u in range(4):
            off = i * 64 + u * 16
            winner[pl.ds(off, 16)] = jnp.bitwise_not(cell_base + off + iota)
        return carry
    lax.fori_loop(0, _CPT // 64, init_body, 0)

    pltpu.async_copy(cell_hbm.at[pl.ds(0, _CHUNK)],
                     cellbuf.at[pl.ds(0, _CHUNK)], sem_in)

    def chunk_body(c, carry):
        slot = lax.rem(c, 2)

        @pl.when(c + 1 < _NCHUNKS)
        def _():
            nslot = lax.rem(c + 1, 2)
            pltpu.async_copy(cell_hbm.at[pl.ds((c + 1) * _CHUNK, _CHUNK)],
                             cellbuf.at[pl.ds(nslot * _CHUNK, _CHUNK)],
                             sem_in)

        pltpu.make_async_copy(cell_hbm.at[pl.ds(c * _CHUNK, _CHUNK)],
                              cellbuf.at[pl.ds(slot * _CHUNK, _CHUNK)],
                              sem_in).wait()

        def vec_body(v, vcarry):
            o0 = v * 128
            cells_l = [cellbuf[pl.ds(slot * _CHUNK + o0 + u * 16, 16)]
                       for u in range(8)]
            for u in range(8):
                o = o0 + u * 16
                local = cells_l[u] - cell_base
                m = (local >= 0) & (local < _CPT)
                lc = jnp.bitwise_and(local, _CPT - 1)
                ivec = c * _CHUNK + o + iota
                plsc.store_scatter(winner, [lc], ivec, mask=m)
            return vcarry
        lax.fori_loop(0, _CHUNK // 128, vec_body, 0)
        return carry
    lax.fori_loop(0, _NCHUNKS, chunk_body, 0)

    def mask_body(i, carry):
        for u in range(8):
            off = i * 128 + u * 16
            w = winner[pl.ds(off, 16)]
            maskvm[pl.ds(off, 16)] = jnp.where(
                w < 0, jnp.float32(0.0), jnp.float32(1.0))
        return carry
    lax.fori_loop(0, _CPT // 128, mask_body, 0)
    pltpu.async_copy(maskvm, mask_out.at[pl.ds(cell_base, _CPT)], sem_mask)

    def prep(gidx, slot):
        blk = lax.shift_right_logical(gidx, 4)
        gg = jnp.bitwise_and(gidx, 15)
        gbase = blk * 2048 + gg * 64
        for half in range(2):
            ib = idxbuf.at[slot, half]
            for v in range(4):
                cl = gbase + half * 1024 + v * 16 + iota
                w = plsc.load_gather(winner, [cl])
                ib[pl.ds(v * 16, 16)] = jnp.where(
                    w < 0, jnp.bitwise_not(w), w)

    def out_dma(gidx, slot, half):
        rb = wid * (_CPT // 2) + gidx * 64
        return pltpu.make_async_copy(
            rowsbuf.at[slot, half],
            table_out.at[pl.ds(rb, 64), pl.ds(half * _C, _C)], sem_rows)

    def gather_dma(gidx, slot, half):
        return pltpu.make_async_copy(
            feats_hbm.at[idxbuf.at[slot, half]],
            rowsbuf.at[slot, half], sem_g)

    def gloop(g, carry):
        slot = lax.rem(g, 2)

        @pl.when(g >= 2)
        def _():
            out_dma(g - 2, slot, 0).wait()
            out_dma(g - 2, slot, 1).wait()

        @pl.when(g < _NG)
        def _():
            prep(g, slot)
            pltpu.async_copy(feats_hbm.at[idxbuf.at[slot, 0]],
                             rowsbuf.at[slot, 0], sem_g)
            pltpu.async_copy(feats_hbm.at[idxbuf.at[slot, 1]],
                             rowsbuf.at[slot, 1], sem_g)

        @pl.when(g >= 1)
        def _():
            pslot = lax.rem(g - 1, 2)
            gather_dma(g - 1, pslot, 0).wait()
            gather_dma(g - 1, pslot, 1).wait()
            rb = wid * (_CPT // 2) + (g - 1) * 64
            pltpu.async_copy(
                rowsbuf.at[pslot, 0],
                table_out.at[pl.ds(rb, 64), pl.ds(0, _C)], sem_rows)
            pltpu.async_copy(
                rowsbuf.at[pslot, 1],
                table_out.at[pl.ds(rb, 64), pl.ds(_C, _C)], sem_rows)
        return carry
    lax.fori_loop(0, _NG + 1, gloop, 0)

    lslot = (_NG - 1) % 2
    out_dma(_NG - 1, lslot, 0).wait()
    out_dma(_NG - 1, lslot, 1).wait()
    pltpu.make_async_copy(maskvm, mask_out.at[pl.ds(cell_base, _CPT)],
                          sem_mask).wait()


_sc_call = pl.kernel(
    _sc_body,
    mesh=plsc.VectorSubcoreMesh(core_axis_name="c", subcore_axis_name="s"),
    compiler_params=pltpu.CompilerParams(
        needs_layout_passes=False, use_tc_tiling_on_sc=False),
    out_type=(
        jax.ShapeDtypeStruct((_M // 2, 2 * _C), jnp.float32),
        jax.ShapeDtypeStruct((_M,), jnp.float32),
    ),
    scratch_types=[
        pltpu.VMEM((2 * _CHUNK,), jnp.int32),
        pltpu.VMEM((_CPT,), jnp.int32),
        pltpu.VMEM((2, 2, 64), jnp.int32),
        pltpu.VMEM((2, 2, 64, _C), jnp.float32),
        pltpu.VMEM((_CPT,), jnp.float32),
        pltpu.SemaphoreType.DMA,
        pltpu.SemaphoreType.DMA,
        pltpu.SemaphoreType.DMA,
        pltpu.SemaphoreType.DMA,
    ],
)


def _tc_body(x_ref, m_ref, o_ref):
    x = x_ref[0]
    t = jnp.transpose(x)
    m2 = m_ref[0, 0]
    y = jnp.concatenate(
        [t[:_C] * m2[0:1], t[_C:] * m2[1:2]], axis=1)
    for xs in range(8):
        o_ref[0, :, xs, :] = y[:, xs * 256:(xs + 1) * 256]


@jax.jit
def _tc_call(table3, mask4):
    return pl.pallas_call(
        _tc_body,
        grid=(_B, _H // 8),
        in_specs=[
            pl.BlockSpec((1, 1024, 2 * _C), lambda b, h: (b, h, 0)),
            pl.BlockSpec((1, 1, 2, 1024), lambda b, h: (b, h, 0, 0)),
        ],
        out_specs=pl.BlockSpec((1, _C, 8, _W), lambda b, h: (b, 0, h, 0)),
        out_shape=jax.ShapeDtypeStruct((_B, _C, _H, _W), jnp.float32),
    )(table3, mask4)


def kernel(feats, batch_idx, coords):
    cell = (batch_idx * (_H * _W)
            + coords[:, 0] * _W + coords[:, 1]).astype(jnp.int32)
    table, maskv = _sc_call(cell, feats)
    return _tc_call(table.reshape(_B, (_H // 8) * 1024, 2 * _C),
                    maskv.reshape(_B, _H // 8, 2, 1024))


# speedup vs baseline: 7.5825x; 1.2025x over previous
import functools

import jax
import jax.numpy as jnp
from jax import lax
from jax.experimental import pallas as pl
from jax.experimental.pallas import tpu as pltpu
from jax.experimental.pallas import tpu_sc as plsc

_B, _C, _H, _W = 4, 64, 256, 256
_N = 262144
_M = _B * _H * _W
_NC, _NS = 2, 16
_NW = _NC * _NS
_CPT = _M // _NW
_CHUNK = 2048
_NCHUNKS = _N // _CHUNK
_GC = 128
_NG = _CPT // _GC


def _sc_body(cell_hbm, feats_hbm, table_out, mask_out,
             cellbuf, winner, idxbuf, rowsbuf, maskvm,
             sem_in, sem_g, sem_rows, sem_mask):
    wid = lax.axis_index("s") * _NC + lax.axis_index("c")
    cell_base = wid * _CPT
    iota = lax.iota(jnp.int32, 16)

    def init_body(i, carry):
        for u in range(4):
            off = i * 64 + u * 16
            winner[pl.ds(off, 16)] = jnp.bitwise_not(cell_base + off + iota)
        return carry
    lax.fori_loop(0, _CPT // 64, init_body, 0)

    pltpu.async_copy(cell_hbm.at[pl.ds(0, _CHUNK)],
                     cellbuf.at[pl.ds(0, _CHUNK)], sem_in)

    def chunk_body(c, carry):
        slot = lax.rem(c, 2)

        @pl.when(c + 1 < _NCHUNKS)
        def _():
            nslot = lax.rem(c + 1, 2)
            pltpu.async_copy(cell_hbm.at[pl.ds((c + 1) * _CHUNK, _CHUNK)],
                             cellbuf.at[pl.ds(nslot * _CHUNK, _CHUNK)],
                             sem_in)

        pltpu.make_async_copy(cell_hbm.at[pl.ds(c * _CHUNK, _CHUNK)],
                              cellbuf.at[pl.ds(slot * _CHUNK, _CHUNK)],
                              sem_in).wait()

        def vec_body(v, vcarry):
            o0 = v * 128
            cells_l = [cellbuf[pl.ds(slot * _CHUNK + o0 + u * 16, 16)]
                       for u in range(8)]
            for u in range(8):
                o = o0 + u * 16
                local = cells_l[u] - cell_base
                m = (local >= 0) & (local < _CPT)
                lc = jnp.bitwise_and(local, _CPT - 1)
                ivec = c * _CHUNK + o + iota
                plsc.store_scatter(winner, [lc], ivec, mask=m)
            return vcarry
        lax.fori_loop(0, _CHUNK // 128, vec_body, 0)
        return carry
    lax.fori_loop(0, _NCHUNKS, chunk_body, 0)

    def mask_body(i, carry):
        mrow = maskvm.at[i]
        for u in range(8):
            w = winner[pl.ds(i * 128 + u * 16, 16)]
            mrow[pl.ds(u * 16, 16)] = jnp.where(
                w < 0, jnp.float32(0.0), jnp.float32(1.0))
        return carry
    lax.fori_loop(0, _CPT // 128, mask_body, 0)
    pltpu.async_copy(maskvm, mask_out.at[pl.ds(wid * 64, 64)], sem_mask)

    def prep(gidx, slot):
        blk = lax.shift_right_logical(gidx, 4)
        gg = jnp.bitwise_and(gidx, 15)
        gbase = blk * 2048 + gg * 64
        for half in range(2):
            ib = idxbuf.at[slot, half]
            for v in range(4):
                cl = gbase + half * 1024 + v * 16 + iota
                w = plsc.load_gather(winner, [cl])
                idx = jnp.where(w < 0, jnp.bitwise_not(w), w)
                ib[pl.ds(v * 16, 16)] = (
                    lax.shift_left(jnp.bitwise_and(idx, _N // 2 - 1), 1)
                    | lax.shift_right_logical(idx, 17))

    def out_dma(gidx, slot, half):
        rb = wid * (_CPT // 2) + gidx * 64
        return pltpu.make_async_copy(
            rowsbuf.at[slot, half],
            table_out.at[pl.ds(rb, 64), pl.ds(half * _C, _C)], sem_rows)

    def gather_dma(gidx, slot, half):
        return pltpu.make_async_copy(
            feats_hbm.at[idxbuf.at[slot, half]],
            rowsbuf.at[slot, half], sem_g)

    def gloop(g, carry):
        slot = lax.rem(g, 2)

        @pl.when(g >= 2)
        def _():
            out_dma(g - 2, slot, 0).wait()
            out_dma(g - 2, slot, 1).wait()

        @pl.when(g < _NG)
        def _():
            prep(g, slot)
            pltpu.async_copy(feats_hbm.at[idxbuf.at[slot, 0]],
                             rowsbuf.at[slot, 0], sem_g)
            pltpu.async_copy(feats_hbm.at[idxbuf.at[slot, 1]],
                             rowsbuf.at[slot, 1], sem_g)

        @pl.when(g >= 1)
        def _():
            pslot = lax.rem(g - 1, 2)
            gather_dma(g - 1, pslot, 0).wait()
            gather_dma(g - 1, pslot, 1).wait()
            rb = wid * (_CPT // 2) + (g - 1) * 64
            pltpu.async_copy(
                rowsbuf.at[pslot, 0],
                table_out.at[pl.ds(rb, 64), pl.ds(0, _C)], sem_rows)
            pltpu.async_copy(
                rowsbuf.at[pslot, 1],
                table_out.at[pl.ds(rb, 64), pl.ds(_C, _C)], sem_rows)
        return carry
    lax.fori_loop(0, _NG + 1, gloop, 0)

    lslot = (_NG - 1) % 2
    out_dma(_NG - 1, lslot, 0).wait()
    out_dma(_NG - 1, lslot, 1).wait()
    pltpu.make_async_copy(maskvm, mask_out.at[pl.ds(wid * 64, 64)],
                          sem_mask).wait()


_sc_call = pl.kernel(
    _sc_body,
    mesh=plsc.VectorSubcoreMesh(core_axis_name="c", subcore_axis_name="s"),
    compiler_params=pltpu.CompilerParams(
        needs_layout_passes=False, use_tc_tiling_on_sc=False),
    out_type=(
        jax.ShapeDtypeStruct((_M // 2, 2 * _C), jnp.float32),
        jax.ShapeDtypeStruct((_M // 128, 128), jnp.float32),
    ),
    scratch_types=[
        pltpu.VMEM((2 * _CHUNK,), jnp.int32),
        pltpu.VMEM((_CPT,), jnp.int32),
        pltpu.VMEM((2, 2, 64), jnp.int32),
        pltpu.VMEM((2, 2, 64, _C), jnp.float32),
        pltpu.VMEM((64, 128), jnp.float32),
        pltpu.SemaphoreType.DMA,
        pltpu.SemaphoreType.DMA,
        pltpu.SemaphoreType.DMA,
        pltpu.SemaphoreType.DMA,
    ],
)


def _tc_prep_body(a_ref, b_ref, o_ref):
    ta = jnp.transpose(a_ref[...])
    tb = jnp.transpose(b_ref[...])
    o_ref[...] = jnp.concatenate([ta, tb], axis=1)


@jax.jit
def _tc_prep(featsT):
    nblk = _N // 2 // 2048
    return pl.pallas_call(
        _tc_prep_body,
        grid=(nblk,),
        in_specs=[
            pl.BlockSpec((_C, 2048), lambda i: (0, i)),
            pl.BlockSpec((_C, 2048), lambda i: (0, i + nblk)),
        ],
        out_specs=pl.BlockSpec((2048, 128), lambda i: (i, 0)),
        out_shape=jax.ShapeDtypeStruct((_N // 2, 128), jnp.float32),
    )(featsT, featsT)


def _tc_body(x_ref, m_ref, o_ref):
    x = x_ref[0]
    t = jnp.transpose(x)
    m = m_ref[...].reshape(1, 2048)
    y = jnp.concatenate(
        [t[:_C] * m[:, :1024], t[_C:] * m[:, 1024:]], axis=1)
    for xs in range(8):
        o_ref[0, :, xs, :] = y[:, xs * 256:(xs + 1) * 256]


@jax.jit
def _tc_call(table3, maskm):
    return pl.pallas_call(
        _tc_body,
        grid=(_B, _H // 8),
        in_specs=[
            pl.BlockSpec((1, 1024, 2 * _C), lambda b, h: (b, h, 0)),
            pl.BlockSpec((16, 128), lambda b, h: (b * (_H // 8) + h, 0)),
        ],
        out_specs=pl.BlockSpec((1, _C, 8, _W), lambda b, h: (b, 0, h, 0)),
        out_shape=jax.ShapeDtypeStruct((_B, _C, _H, _W), jnp.float32),
    )(table3, maskm)


def kernel(feats, batch_idx, coords):
    cell = (batch_idx * (_H * _W)
            + coords[:, 0] * _W + coords[:, 1]).astype(jnp.int32)
    feats_lin = _tc_prep(feats.T).reshape(_N, _C)
    table, maskm = _sc_call(cell, feats_lin)
    return _tc_call(table.reshape(_B, (_H // 8) * 1024, 2 * _C), maskm)


# speedup vs baseline: 8.8654x; 1.1692x over previous
import functools

import jax
import jax.numpy as jnp
from jax import lax
from jax.experimental import pallas as pl
from jax.experimental.pallas import tpu as pltpu
from jax.experimental.pallas import tpu_sc as plsc

_B, _C, _H, _W = 4, 64, 256, 256
_N = 262144
_M = _B * _H * _W
_NC, _NS = 2, 16
_NW = _NC * _NS
_CPT = _M // _NW
_CHUNK = 2048
_NCHUNKS = _N // _CHUNK
_GC = 128
_NG = _CPT // _GC


def _sc_a_body(cell_hbm, winner_out, mask_out,
               cellbuf, winner, maskvm, sem_in, sem_w, sem_mask):
    wid = lax.axis_index("s") * _NC + lax.axis_index("c")
    cell_base = wid * _CPT
    iota = lax.iota(jnp.int32, 16)

    def init_body(i, carry):
        for u in range(4):
            off = i * 64 + u * 16
            winner[pl.ds(off, 16)] = jnp.bitwise_not(cell_base + off + iota)
        return carry
    lax.fori_loop(0, _CPT // 64, init_body, 0)

    pltpu.async_copy(cell_hbm.at[pl.ds(0, _CHUNK)],
                     cellbuf.at[pl.ds(0, _CHUNK)], sem_in)

    def chunk_body(c, carry):
        slot = lax.rem(c, 2)

        @pl.when(c + 1 < _NCHUNKS)
        def _():
            nslot = lax.rem(c + 1, 2)
            pltpu.async_copy(cell_hbm.at[pl.ds((c + 1) * _CHUNK, _CHUNK)],
                             cellbuf.at[pl.ds(nslot * _CHUNK, _CHUNK)],
                             sem_in)

        pltpu.make_async_copy(cell_hbm.at[pl.ds(c * _CHUNK, _CHUNK)],
                              cellbuf.at[pl.ds(slot * _CHUNK, _CHUNK)],
                              sem_in).wait()

        def vec_body(v, vcarry):
            o0 = v * 128
            cells_l = [cellbuf[pl.ds(slot * _CHUNK + o0 + u * 16, 16)]
                       for u in range(8)]
            for u in range(8):
                o = o0 + u * 16
                local = cells_l[u] - cell_base
                m = (local >= 0) & (local < _CPT)
                lc = jnp.bitwise_and(local, _CPT - 1)
                ivec = c * _CHUNK + o + iota
                plsc.store_scatter(winner, [lc], ivec, mask=m)
            return vcarry
        lax.fori_loop(0, _CHUNK // 128, vec_body, 0)
        return carry
    lax.fori_loop(0, _NCHUNKS, chunk_body, 0)

    def mask_body(i, carry):
        mrow = maskvm.at[i]
        for u in range(8):
            w = winner[pl.ds(i * 128 + u * 16, 16)]
            mrow[pl.ds(u * 16, 16)] = jnp.where(
                w < 0, jnp.float32(0.0), jnp.float32(1.0))
        return carry
    lax.fori_loop(0, _CPT // 128, mask_body, 0)
    pltpu.async_copy(maskvm, mask_out.at[pl.ds(wid * 64, 64)], sem_mask)
    pltpu.async_copy(winner, winner_out.at[pl.ds(cell_base, _CPT)], sem_w)
    pltpu.make_async_copy(maskvm, mask_out.at[pl.ds(wid * 64, 64)],
                          sem_mask).wait()
    pltpu.make_async_copy(winner, winner_out.at[pl.ds(cell_base, _CPT)],
                          sem_w).wait()


def _sc_b_body(winner_hbm, feats_hbm, table_out,
               winner, idxbuf, rowsbuf, sem_g, sem_rows):
    wid = lax.axis_index("s") * _NC + lax.axis_index("c")
    cell_base = wid * _CPT
    iota = lax.iota(jnp.int32, 16)
    pltpu.sync_copy(winner_hbm.at[pl.ds(cell_base, _CPT)], winner)

    def prep(gidx, slot):
        blk = lax.shift_right_logical(gidx, 4)
        gg = jnp.bitwise_and(gidx, 15)
        gbase = blk * 2048 + gg * 64
        for half in range(2):
            ib = idxbuf.at[slot, half]
            for v in range(4):
                cl = gbase + half * 1024 + v * 16 + iota
                w = plsc.load_gather(winner, [cl])
                idx = jnp.where(w < 0, jnp.bitwise_not(w), w)
                ib[pl.ds(v * 16, 16)] = (
                    lax.shift_left(jnp.bitwise_and(idx, _N // 2 - 1), 1)
                    | lax.shift_right_logical(idx, 17))

    def out_dma(gidx, slot, half):
        rb = wid * (_CPT // 2) + gidx * 64
        return pltpu.make_async_copy(
            rowsbuf.at[slot, half],
            table_out.at[pl.ds(rb, 64), pl.ds(half * _C, _C)], sem_rows)

    def gather_dma(gidx, slot, half):
        return pltpu.make_async_copy(
            feats_hbm.at[idxbuf.at[slot, half]],
            rowsbuf.at[slot, half], sem_g)

    def gloop(g, carry):
        slot = lax.rem(g, 2)

        @pl.when(g >= 2)
        def _():
            out_dma(g - 2, slot, 0).wait()
            out_dma(g - 2, slot, 1).wait()

        @pl.when(g < _NG)
        def _():
            prep(g, slot)
            pltpu.async_copy(feats_hbm.at[idxbuf.at[slot, 0]],
                             rowsbuf.at[slot, 0], sem_g)
            pltpu.async_copy(feats_hbm.at[idxbuf.at[slot, 1]],
                             rowsbuf.at[slot, 1], sem_g)

        @pl.when(g >= 1)
        def _():
            pslot = lax.rem(g - 1, 2)
            gather_dma(g - 1, pslot, 0).wait()
            gather_dma(g - 1, pslot, 1).wait()
            rb = wid * (_CPT // 2) + (g - 1) * 64
            pltpu.async_copy(
                rowsbuf.at[pslot, 0],
                table_out.at[pl.ds(rb, 64), pl.ds(0, _C)], sem_rows)
            pltpu.async_copy(
                rowsbuf.at[pslot, 1],
                table_out.at[pl.ds(rb, 64), pl.ds(_C, _C)], sem_rows)
        return carry
    lax.fori_loop(0, _NG + 1, gloop, 0)

    lslot = (_NG - 1) % 2
    out_dma(_NG - 1, lslot, 0).wait()
    out_dma(_NG - 1, lslot, 1).wait()


_sc_mesh = plsc.VectorSubcoreMesh(core_axis_name="c", subcore_axis_name="s")
_sc_params = pltpu.CompilerParams(
    needs_layout_passes=False, use_tc_tiling_on_sc=False)

_sc_a_call = pl.kernel(
    _sc_a_body,
    mesh=_sc_mesh,
    compiler_params=_sc_params,
    out_type=(
        jax.ShapeDtypeStruct((_M,), jnp.int32),
        jax.ShapeDtypeStruct((_M // 128, 128), jnp.float32),
    ),
    scratch_types=[
        pltpu.VMEM((2 * _CHUNK,), jnp.int32),
        pltpu.VMEM((_CPT,), jnp.int32),
        pltpu.VMEM((64, 128), jnp.float32),
        pltpu.SemaphoreType.DMA,
        pltpu.SemaphoreType.DMA,
        pltpu.SemaphoreType.DMA,
    ],
)

_sc_b_call = pl.kernel(
    _sc_b_body,
    mesh=_sc_mesh,
    compiler_params=_sc_params,
    out_type=jax.ShapeDtypeStruct((_M // 2, 2 * _C), jnp.float32),
    scratch_types=[
        pltpu.VMEM((_CPT,), jnp.int32),
        pltpu.VMEM((2, 2, 64), jnp.int32),
        pltpu.VMEM((2, 2, 64, _C), jnp.float32),
        pltpu.SemaphoreType.DMA,
        pltpu.SemaphoreType.DMA,
    ],
)


def _tc_prep_body(a_ref, b_ref, o_ref):
    ta = jnp.transpose(a_ref[...])
    tb = jnp.transpose(b_ref[...])
    o_ref[...] = jnp.concatenate([ta, tb], axis=1)


@jax.jit
def _tc_prep(featsT):
    nblk = _N // 2 // 2048
    return pl.pallas_call(
        _tc_prep_body,
        grid=(nblk,),
        in_specs=[
            pl.BlockSpec((_C, 2048), lambda i: (0, i)),
            pl.BlockSpec((_C, 2048), lambda i: (0, i + nblk)),
        ],
        out_specs=pl.BlockSpec((2048, 128), lambda i: (i, 0)),
        out_shape=jax.ShapeDtypeStruct((_N // 2, 128), jnp.float32),
    )(featsT, featsT)


def _tc_body(x_ref, m_ref, o_ref):
    x = x_ref[0]
    t = jnp.transpose(x)
    m = m_ref[...].reshape(1, 2048)
    y = jnp.concatenate(
        [t[:_C] * m[:, :1024], t[_C:] * m[:, 1024:]], axis=1)
    for xs in range(8):
        o_ref[0, :, xs, :] = y[:, xs * 256:(xs + 1) * 256]


@jax.jit
def _tc_call(table3, maskm):
    return pl.pallas_call(
        _tc_body,
        grid=(_B, _H // 8),
        in_specs=[
            pl.BlockSpec((1, 1024, 2 * _C), lambda b, h: (b, h, 0)),
            pl.BlockSpec((16, 128), lambda b, h: (b * (_H // 8) + h, 0)),
        ],
        out_specs=pl.BlockSpec((1, _C, 8, _W), lambda b, h: (b, 0, h, 0)),
        out_shape=jax.ShapeDtypeStruct((_B, _C, _H, _W), jnp.float32),
    )(table3, maskm)


def kernel(feats, batch_idx, coords):
    cell = (batch_idx * (_H * _W)
            + coords[:, 0] * _W + coords[:, 1]).astype(jnp.int32)
    feats_lin = _tc_prep(feats.T).reshape(_N, _C)
    winner, maskm = _sc_a_call(cell)
    table = _sc_b_call(winner, feats_lin)
    return _tc_call(table.reshape(_B, (_H // 8) * 1024, 2 * _C), maskm)


# speedup vs baseline: 10.3982x; 1.1729x over previous
import functools

import jax
import jax.numpy as jnp
from jax import lax
from jax.experimental import pallas as pl
from jax.experimental.pallas import tpu as pltpu
from jax.experimental.pallas import tpu_sc as plsc

_B, _C, _H, _W = 4, 64, 256, 256
_N = 262144
_M = _B * _H * _W
_NC, _NS = 2, 16
_NW = _NC * _NS
_CPT = _M // _NW
_CHUNK = 2048
_NCHUNKS = _N // _CHUNK
_GC = 128
_NG = _CPT // _GC


_CROWS = _CHUNK // 128


def _sc_a_body(cell_hbm, winner_out, mask_out,
               cellbuf, winner, maskvm, sem_in, sem_w, sem_mask):
    wid = lax.axis_index("s") * _NC + lax.axis_index("c")
    cell_base = wid * _CPT
    iota = lax.iota(jnp.int32, 16)

    def init_body(i, carry):
        for u in range(4):
            off = i * 64 + u * 16
            winner[pl.ds(off, 16)] = jnp.bitwise_not(cell_base + off + iota)
        return carry
    lax.fori_loop(0, _CPT // 64, init_body, 0)

    pltpu.async_copy(cell_hbm.at[pl.ds(0, _CROWS)],
                     cellbuf.at[pl.ds(0, _CROWS)], sem_in)

    def chunk_body(c, carry):
        slot = lax.rem(c, 2)

        @pl.when(c + 1 < _NCHUNKS)
        def _():
            nslot = lax.rem(c + 1, 2)
            pltpu.async_copy(
                cell_hbm.at[pl.ds((c + 1) * _CROWS, _CROWS)],
                cellbuf.at[pl.ds(nslot * _CROWS, _CROWS)], sem_in)

        pltpu.make_async_copy(cell_hbm.at[pl.ds(c * _CROWS, _CROWS)],
                              cellbuf.at[pl.ds(slot * _CROWS, _CROWS)],
                              sem_in).wait()

        def vec_body(v, vcarry):
            crow = cellbuf.at[slot * _CROWS + v]
            cells_l = [crow[pl.ds(u * 16, 16)] for u in range(8)]
            for u in range(8):
                o = v * 128 + u * 16
                local = cells_l[u] - cell_base
                m = (local >= 0) & (local < _CPT)
                lc = jnp.bitwise_and(local, _CPT - 1)
                ivec = c * _CHUNK + o + iota
                plsc.store_scatter(winner, [lc], ivec, mask=m)
            return vcarry
        lax.fori_loop(0, _CROWS, vec_body, 0)
        return carry
    lax.fori_loop(0, _NCHUNKS, chunk_body, 0)

    def mask_body(i, carry):
        mrow = maskvm.at[i]
        for u in range(8):
            w = winner[pl.ds(i * 128 + u * 16, 16)]
            mrow[pl.ds(u * 16, 16)] = jnp.where(
                w < 0, jnp.float32(0.0), jnp.float32(1.0))
        return carry
    lax.fori_loop(0, _CPT // 128, mask_body, 0)
    pltpu.async_copy(maskvm, mask_out.at[pl.ds(wid * 64, 64)], sem_mask)
    pltpu.async_copy(winner, winner_out.at[pl.ds(cell_base, _CPT)], sem_w)
    pltpu.make_async_copy(maskvm, mask_out.at[pl.ds(wid * 64, 64)],
                          sem_mask).wait()
    pltpu.make_async_copy(winner, winner_out.at[pl.ds(cell_base, _CPT)],
                          sem_w).wait()


def _sc_b_body(winner_hbm, feats_hbm, table_out,
               winner, idxbuf, rowsbuf, sem_g, sem_rows):
    wid = lax.axis_index("s") * _NC + lax.axis_index("c")
    cell_base = wid * _CPT
    iota = lax.iota(jnp.int32, 16)
    pltpu.sync_copy(winner_hbm.at[pl.ds(cell_base, _CPT)], winner)

    def prep(gidx, slot):
        blk = lax.shift_right_logical(gidx, 4)
        gg = jnp.bitwise_and(gidx, 15)
        gbase = blk * 2048 + gg * 64
        for half in range(2):
            ib = idxbuf.at[slot, half]
            for v in range(4):
                cl = gbase + half * 1024 + v * 16 + iota
                w = plsc.load_gather(winner, [cl])
                idx = jnp.where(w < 0, jnp.bitwise_not(w), w)
                ib[pl.ds(v * 16, 16)] = (
                    lax.shift_left(jnp.bitwise_and(idx, _N // 2 - 1), 1)
                    | lax.shift_right_logical(idx, 17))

    def out_dma(gidx, slot, half):
        rb = wid * (_CPT // 2) + gidx * 64
        return pltpu.make_async_copy(
            rowsbuf.at[slot, half],
            table_out.at[pl.ds(rb, 64), pl.ds(half * _C, _C)], sem_rows)

    def gather_dma(gidx, slot, half):
        return pltpu.make_async_copy(
            feats_hbm.at[idxbuf.at[slot, half]],
            rowsbuf.at[slot, half], sem_g)

    def gloop(g, carry):
        slot = lax.rem(g, 2)

        @pl.when(g >= 2)
        def _():
            out_dma(g - 2, slot, 0).wait()
            out_dma(g - 2, slot, 1).wait()

        @pl.when(g < _NG)
        def _():
            prep(g, slot)
            pltpu.async_copy(feats_hbm.at[idxbuf.at[slot, 0]],
                             rowsbuf.at[slot, 0], sem_g)
            pltpu.async_copy(feats_hbm.at[idxbuf.at[slot, 1]],
                             rowsbuf.at[slot, 1], sem_g)

        @pl.when(g >= 1)
        def _():
            pslot = lax.rem(g - 1, 2)
            gather_dma(g - 1, pslot, 0).wait()
            gather_dma(g - 1, pslot, 1).wait()
            rb = wid * (_CPT // 2) + (g - 1) * 64
            pltpu.async_copy(
                rowsbuf.at[pslot, 0],
                table_out.at[pl.ds(rb, 64), pl.ds(0, _C)], sem_rows)
            pltpu.async_copy(
                rowsbuf.at[pslot, 1],
                table_out.at[pl.ds(rb, 64), pl.ds(_C, _C)], sem_rows)
        return carry
    lax.fori_loop(0, _NG + 1, gloop, 0)

    lslot = (_NG - 1) % 2
    out_dma(_NG - 1, lslot, 0).wait()
    out_dma(_NG - 1, lslot, 1).wait()


_sc_mesh = plsc.VectorSubcoreMesh(core_axis_name="c", subcore_axis_name="s")
_sc_params = pltpu.CompilerParams(
    needs_layout_passes=False, use_tc_tiling_on_sc=False)

_sc_a_call = pl.kernel(
    _sc_a_body,
    mesh=_sc_mesh,
    compiler_params=_sc_params,
    out_type=(
        jax.ShapeDtypeStruct((_M,), jnp.int32),
        jax.ShapeDtypeStruct((_M // 128, 128), jnp.float32),
    ),
    scratch_types=[
        pltpu.VMEM((2 * _CROWS, 128), jnp.int32),
        pltpu.VMEM((_CPT,), jnp.int32),
        pltpu.VMEM((64, 128), jnp.float32),
        pltpu.SemaphoreType.DMA,
        pltpu.SemaphoreType.DMA,
        pltpu.SemaphoreType.DMA,
    ],
)

_sc_b_call = pl.kernel(
    _sc_b_body,
    mesh=_sc_mesh,
    compiler_params=_sc_params,
    out_type=jax.ShapeDtypeStruct((_M // 2, 2 * _C), jnp.float32),
    scratch_types=[
        pltpu.VMEM((_CPT,), jnp.int32),
        pltpu.VMEM((2, 2, 64), jnp.int32),
        pltpu.VMEM((2, 2, 64, _C), jnp.float32),
        pltpu.SemaphoreType.DMA,
        pltpu.SemaphoreType.DMA,
    ],
)


def _tc_prep_body(a_ref, b_ref, o_ref):
    ta = jnp.transpose(a_ref[...])
    tb = jnp.transpose(b_ref[...])
    o_ref[...] = jnp.concatenate([ta, tb], axis=1)


@jax.jit
def _tc_prep(featsT):
    nblk = _N // 2 // 8192
    return pl.pallas_call(
        _tc_prep_body,
        grid=(nblk,),
        in_specs=[
            pl.BlockSpec((_C, 8192), lambda i: (0, i)),
            pl.BlockSpec((_C, 8192), lambda i: (0, i + nblk)),
        ],
        out_specs=pl.BlockSpec((8192, 128), lambda i: (i, 0)),
        out_shape=jax.ShapeDtypeStruct((_N // 2, 128), jnp.float32),
    )(featsT, featsT)


def _tc_body(x_ref, m_ref, o_ref):
    for q in range(4):
        x = x_ref[0, pl.ds(q * 1024, 1024), :]
        t = jnp.transpose(x)
        m = m_ref[pl.ds(q * 16, 16), :].reshape(1, 2048)
        y = jnp.concatenate(
            [t[:_C] * m[:, :1024], t[_C:] * m[:, 1024:]], axis=1)
        for xs in range(8):
            o_ref[0, :, q * 8 + xs, :] = y[:, xs * 256:(xs + 1) * 256]


@jax.jit
def _tc_call(table3, maskm):
    return pl.pallas_call(
        _tc_body,
        grid=(_B, _H // 32),
        in_specs=[
            pl.BlockSpec((1, 4096, 2 * _C), lambda b, h: (b, h, 0)),
            pl.BlockSpec((64, 128), lambda b, h: (b * (_H // 32) + h, 0)),
        ],
        out_specs=pl.BlockSpec((1, _C, 32, _W), lambda b, h: (b, 0, h, 0)),
        out_shape=jax.ShapeDtypeStruct((_B, _C, _H, _W), jnp.float32),
    )(table3, maskm)


def kernel(feats, batch_idx, coords):
    cell = (batch_idx * (_H * _W)
            + coords[:, 0] * _W + coords[:, 1]).astype(jnp.int32)
    feats_lin = _tc_prep(feats.T).reshape(_N, _C)
    winner, maskm = _sc_a_call(cell.reshape(_N // 128, 128))
    table = _sc_b_call(winner, feats_lin)
    return _tc_call(table.reshape(_B, (_H // 8) * 1024, 2 * _C), maskm)


# speedup vs baseline: 12.3078x; 1.1836x over previous
import functools

import jax
import jax.numpy as jnp
from jax import lax
from jax.experimental import pallas as pl
from jax.experimental.pallas import tpu as pltpu
from jax.experimental.pallas import tpu_sc as plsc

_B, _C, _H, _W = 4, 64, 256, 256
_N = 262144
_M = _B * _H * _W
_NC, _NS = 2, 16
_NW = _NC * _NS
_CPT = _M // _NW
_CHUNK = 2048
_NCHUNKS = _N // _CHUNK
_GC = 128
_NG = _CPT // _GC


_CROWS = _CHUNK // 128


def _sc_a_body(cell_hbm, winner_out, mask_out,
               cellbuf, winner, maskvm, cellsh, sem_in, sem_w, sem_mask):
    wid = lax.axis_index("s") * _NC + lax.axis_index("c")
    sid = lax.axis_index("s")
    cell_base = wid * _CPT
    iota = lax.iota(jnp.int32, 16)

    pltpu.async_copy(cell_hbm.at[pl.ds(sid * 128, 128)],
                     cellsh.at[pl.ds(sid * 128, 128)], sem_in)

    def init_body(i, carry):
        for u in range(4):
            off = i * 64 + u * 16
            winner[pl.ds(off, 16)] = jnp.bitwise_not(cell_base + off + iota)
        return carry
    lax.fori_loop(0, _CPT // 64, init_body, 0)

    pltpu.make_async_copy(cell_hbm.at[pl.ds(sid * 128, 128)],
                          cellsh.at[pl.ds(sid * 128, 128)], sem_in).wait()
    plsc.subcore_barrier()
    pltpu.async_copy(cellsh.at[pl.ds(0, _CROWS)],
                     cellbuf.at[pl.ds(0, _CROWS)], sem_in)

    def chunk_body(c, carry):
        slot = lax.rem(c, 2)

        @pl.when(c + 1 < _NCHUNKS)
        def _():
            nslot = lax.rem(c + 1, 2)
            pltpu.async_copy(
                cellsh.at[pl.ds((c + 1) * _CROWS, _CROWS)],
                cellbuf.at[pl.ds(nslot * _CROWS, _CROWS)], sem_in)

        pltpu.make_async_copy(cellsh.at[pl.ds(c * _CROWS, _CROWS)],
                              cellbuf.at[pl.ds(slot * _CROWS, _CROWS)],
                              sem_in).wait()

        def vec_body(v, vcarry):
            crow = cellbuf.at[slot * _CROWS + v]
            cells_l = [crow[pl.ds(u * 16, 16)] for u in range(8)]
            for u in range(8):
                o = v * 128 + u * 16
                local = cells_l[u] - cell_base
                m = (local >= 0) & (local < _CPT)
                lc = jnp.bitwise_and(local, _CPT - 1)
                ivec = c * _CHUNK + o + iota
                plsc.store_scatter(winner, [lc], ivec, mask=m)
            return vcarry
        lax.fori_loop(0, _CROWS, vec_body, 0)
        return carry
    lax.fori_loop(0, _NCHUNKS, chunk_body, 0)

    def mask_body(i, carry):
        mrow = maskvm.at[i]
        for u in range(8):
            w = winner[pl.ds(i * 128 + u * 16, 16)]
            mrow[pl.ds(u * 16, 16)] = jnp.where(
                w < 0, jnp.float32(0.0), jnp.float32(1.0))
        return carry
    lax.fori_loop(0, _CPT // 128, mask_body, 0)
    pltpu.async_copy(maskvm, mask_out.at[pl.ds(wid * 64, 64)], sem_mask)
    pltpu.async_copy(winner, winner_out.at[pl.ds(cell_base, _CPT)], sem_w)
    pltpu.make_async_copy(maskvm, mask_out.at[pl.ds(wid * 64, 64)],
                          sem_mask).wait()
    pltpu.make_async_copy(winner, winner_out.at[pl.ds(cell_base, _CPT)],
                          sem_w).wait()


def _sc_b_body(winner_hbm, feats_hbm, table_out,
               winner, idxbuf, rowsbuf, sem_g, sem_rows):
    wid = lax.axis_index("s") * _NC + lax.axis_index("c")
    cell_base = wid * _CPT
    iota = lax.iota(jnp.int32, 16)
    pltpu.sync_copy(winner_hbm.at[pl.ds(cell_base, _CPT)], winner)

    def prep(gidx, slot):
        blk = lax.shift_right_logical(gidx, 4)
        gg = jnp.bitwise_and(gidx, 15)
        gbase = blk * 2048 + gg * 64
        for half in range(2):
            ib = idxbuf.at[slot, half]
            for v in range(4):
                cl = gbase + half * 1024 + v * 16 + iota
                w = plsc.load_gather(winner, [cl])
                idx = jnp.where(w < 0, jnp.bitwise_not(w), w)
                ib[pl.ds(v * 16, 16)] = (
                    lax.shift_left(jnp.bitwise_and(idx, _N // 2 - 1), 1)
                    | lax.shift_right_logical(idx, 17))

    def out_dma(gidx, slot, half):
        rb = wid * (_CPT // 2) + gidx * 64
        return pltpu.make_async_copy(
            rowsbuf.at[slot, half],
            table_out.at[pl.ds(rb, 64), pl.ds(half * _C, _C)], sem_rows)

    def gather_dma(gidx, slot, half):
        return pltpu.make_async_copy(
            feats_hbm.at[idxbuf.at[slot, half]],
            rowsbuf.at[slot, half], sem_g)

    def gloop(g, carry):
        slot = lax.rem(g, 2)

        @pl.when(g >= 2)
        def _():
            out_dma(g - 2, slot, 0).wait()
            out_dma(g - 2, slot, 1).wait()

        @pl.when(g < _NG)
        def _():
            prep(g, slot)
            pltpu.async_copy(feats_hbm.at[idxbuf.at[slot, 0]],
                             rowsbuf.at[slot, 0], sem_g)
            pltpu.async_copy(feats_hbm.at[idxbuf.at[slot, 1]],
                             rowsbuf.at[slot, 1], sem_g)

        @pl.when(g >= 1)
        def _():
            pslot = lax.rem(g - 1, 2)
            gather_dma(g - 1, pslot, 0).wait()
            gather_dma(g - 1, pslot, 1).wait()
            rb = wid * (_CPT // 2) + (g - 1) * 64
            pltpu.async_copy(
                rowsbuf.at[pslot, 0],
                table_out.at[pl.ds(rb, 64), pl.ds(0, _C)], sem_rows)
            pltpu.async_copy(
                rowsbuf.at[pslot, 1],
                table_out.at[pl.ds(rb, 64), pl.ds(_C, _C)], sem_rows)
        return carry
    lax.fori_loop(0, _NG + 1, gloop, 0)

    lslot = (_NG - 1) % 2
    out_dma(_NG - 1, lslot, 0).wait()
    out_dma(_NG - 1, lslot, 1).wait()


_sc_mesh = plsc.VectorSubcoreMesh(core_axis_name="c", subcore_axis_name="s")
_sc_params = pltpu.CompilerParams(
    needs_layout_passes=False, use_tc_tiling_on_sc=False)

_sc_a_call = pl.kernel(
    _sc_a_body,
    mesh=_sc_mesh,
    compiler_params=pltpu.CompilerParams(
        needs_layout_passes=False, use_tc_tiling_on_sc=True),
    out_type=(
        jax.ShapeDtypeStruct((_M,), jnp.int32),
        jax.ShapeDtypeStruct((_M // 128, 128), jnp.float32),
    ),
    scratch_types=[
        pltpu.VMEM((2 * _CROWS, 128), jnp.int32),
        pltpu.VMEM((_CPT,), jnp.int32),
        pltpu.VMEM((64, 128), jnp.float32),
        pltpu.VMEM_SHARED((_N // 128, 128), jnp.int32),
        pltpu.SemaphoreType.DMA,
        pltpu.SemaphoreType.DMA,
        pltpu.SemaphoreType.DMA,
    ],
)

_sc_b_call = pl.kernel(
    _sc_b_body,
    mesh=_sc_mesh,
    compiler_params=_sc_params,
    out_type=jax.ShapeDtypeStruct((_M // 2, 2 * _C), jnp.float32),
    scratch_types=[
        pltpu.VMEM((_CPT,), jnp.int32),
        pltpu.VMEM((2, 2, 64), jnp.int32),
        pltpu.VMEM((2, 2, 64, _C), jnp.float32),
        pltpu.SemaphoreType.DMA,
        pltpu.SemaphoreType.DMA,
    ],
)


def _tc_prep_body(a_ref, b_ref, o_ref):
    ta = jnp.transpose(a_ref[...])
    tb = jnp.transpose(b_ref[...])
    o_ref[...] = jnp.concatenate([ta, tb], axis=1)


@jax.jit
def _tc_prep(featsT):
    nblk = _N // 2 // 8192
    return pl.pallas_call(
        _tc_prep_body,
        grid=(nblk,),
        in_specs=[
            pl.BlockSpec((_C, 8192), lambda i: (0, i)),
            pl.BlockSpec((_C, 8192), lambda i: (0, i + nblk)),
        ],
        out_specs=pl.BlockSpec((8192, 128), lambda i: (i, 0)),
        out_shape=jax.ShapeDtypeStruct((_N // 2, 128), jnp.float32),
    )(featsT, featsT)


def _tc_body(x_ref, m_ref, o_ref):
    for q in range(4):
        x = x_ref[0, pl.ds(q * 1024, 1024), :]
        t = jnp.transpose(x)
        m = m_ref[pl.ds(q * 16, 16), :].reshape(1, 2048)
        y = jnp.concatenate(
            [t[:_C] * m[:, :1024], t[_C:] * m[:, 1024:]], axis=1)
        for xs in range(8):
            o_ref[0, :, q * 8 + xs, :] = y[:, xs * 256:(xs + 1) * 256]


@jax.jit
def _tc_call(table3, maskm):
    return pl.pallas_call(
        _tc_body,
        grid=(_B, _H // 32),
        in_specs=[
            pl.BlockSpec((1, 4096, 2 * _C), lambda b, h: (b, h, 0)),
            pl.BlockSpec((64, 128), lambda b, h: (b * (_H // 32) + h, 0)),
        ],
        out_specs=pl.BlockSpec((1, _C, 32, _W), lambda b, h: (b, 0, h, 0)),
        out_shape=jax.ShapeDtypeStruct((_B, _C, _H, _W), jnp.float32),
    )(table3, maskm)


def kernel(feats, batch_idx, coords):
    cell = (batch_idx * (_H * _W)
            + coords[:, 0] * _W + coords[:, 1]).astype(jnp.int32)
    feats_lin = _tc_prep(feats.T).reshape(_N, _C)
    winner, maskm = _sc_a_call(cell.reshape(_N // 128, 128))
    table = _sc_b_call(winner, feats_lin)
    return _tc_call(table.reshape(_B, (_H // 8) * 1024, 2 * _C), maskm)


# speedup vs baseline: 13.6418x; 1.1084x over previous
import functools

import jax
import jax.numpy as jnp
from jax import lax
from jax.experimental import pallas as pl
from jax.experimental.pallas import tpu as pltpu
from jax.experimental.pallas import tpu_sc as plsc

_B, _C, _H, _W = 4, 64, 256, 256
_N = 262144
_M = _B * _H * _W
_NC, _NS = 2, 16
_NW = _NC * _NS
_CPT = _M // _NW
_CHUNK = 2048
_NCHUNKS = _N // _CHUNK
_GC = 128
_NG = _CPT // _GC


_CROWS = _CHUNK // 128


def _sc_a_body(cell_hbm, winner_out, mask_out,
               cellbuf, winner, maskvm, cellsh, sem_in, sem_w, sem_mask):
    wid = lax.axis_index("s") * _NC + lax.axis_index("c")
    sid = lax.axis_index("s")
    cell_base = wid * _CPT
    iota = lax.iota(jnp.int32, 16)

    pltpu.async_copy(cell_hbm.at[pl.ds(sid * 128, 128)],
                     cellsh.at[pl.ds(sid * 128, 128)], sem_in)

    def init_body(i, carry):
        for u in range(4):
            off = i * 64 + u * 16
            winner[pl.ds(off, 16)] = jnp.bitwise_not(cell_base + off + iota)
        return carry
    lax.fori_loop(0, _CPT // 64, init_body, 0)

    pltpu.make_async_copy(cell_hbm.at[pl.ds(sid * 128, 128)],
                          cellsh.at[pl.ds(sid * 128, 128)], sem_in).wait()
    plsc.subcore_barrier()
    pltpu.async_copy(cellsh.at[pl.ds(0, _CROWS)],
                     cellbuf.at[pl.ds(0, _CROWS)], sem_in)

    def chunk_body(c, carry):
        slot = lax.rem(c, 2)

        @pl.when(c + 1 < _NCHUNKS)
        def _():
            nslot = lax.rem(c + 1, 2)
            pltpu.async_copy(
                cellsh.at[pl.ds((c + 1) * _CROWS, _CROWS)],
                cellbuf.at[pl.ds(nslot * _CROWS, _CROWS)], sem_in)

        pltpu.make_async_copy(cellsh.at[pl.ds(c * _CROWS, _CROWS)],
                              cellbuf.at[pl.ds(slot * _CROWS, _CROWS)],
                              sem_in).wait()

        def vec_body(v, vcarry):
            crow = cellbuf.at[slot * _CROWS + v]
            cells_l = [crow[pl.ds(u * 16, 16)] for u in range(8)]
            for u in range(8):
                o = v * 128 + u * 16
                local = cells_l[u] - cell_base
                m = (local >= 0) & (local < _CPT)
                lc = jnp.bitwise_and(local, _CPT - 1)
                ivec = c * _CHUNK + o + iota
                plsc.store_scatter(winner, [lc], ivec, mask=m)
            return vcarry
        lax.fori_loop(0, _CROWS, vec_body, 0)
        return carry
    lax.fori_loop(0, _NCHUNKS, chunk_body, 0)

    def mask_body(i, carry):
        mrow = maskvm.at[i]
        for u in range(8):
            w = winner[pl.ds(i * 128 + u * 16, 16)]
            mrow[pl.ds(u * 16, 16)] = jnp.where(
                w < 0, jnp.float32(0.0), jnp.float32(1.0))
        return carry
    lax.fori_loop(0, _CPT // 128, mask_body, 0)
    pltpu.async_copy(maskvm, mask_out.at[pl.ds(wid * 64, 64)], sem_mask)
    pltpu.async_copy(winner, winner_out.at[pl.ds(cell_base, _CPT)], sem_w)
    pltpu.make_async_copy(maskvm, mask_out.at[pl.ds(wid * 64, 64)],
                          sem_mask).wait()
    pltpu.make_async_copy(winner, winner_out.at[pl.ds(cell_base, _CPT)],
                          sem_w).wait()


def _sc_b_body(winner_hbm, feats_hbm, table_out,
               winner, idxbuf, rowsbuf, sem_g, sem_rows):
    wid = lax.axis_index("s") * _NC + lax.axis_index("c")
    cell_base = wid * _CPT
    iota = lax.iota(jnp.int32, 16)
    pltpu.sync_copy(winner_hbm.at[pl.ds(cell_base, _CPT)], winner)

    def prep(gidx, slot):
        blk = lax.shift_right_logical(gidx, 4)
        gg = jnp.bitwise_and(gidx, 15)
        gbase = blk * 2048 + gg * 64
        for half in range(2):
            ib = idxbuf.at[slot, half]
            for v in range(4):
                cl = gbase + half * 1024 + v * 16 + iota
                w = plsc.load_gather(winner, [cl])
                idx = jnp.where(w < 0, jnp.bitwise_not(w), w)
                ib[pl.ds(v * 16, 16)] = (
                    lax.shift_left(jnp.bitwise_and(idx, _N // 2 - 1), 1)
                    | lax.shift_right_logical(idx, 17))

    def out_dma(gidx, slot, half):
        rb = wid * (_CPT // 2) + gidx * 64
        return pltpu.make_async_copy(
            rowsbuf.at[slot, half],
            table_out.at[pl.ds(rb, 64), pl.ds(half * _C, _C)], sem_rows)

    def gather_dma(gidx, slot, half):
        return pltpu.make_async_copy(
            feats_hbm.at[idxbuf.at[slot, half]],
            rowsbuf.at[slot, half], sem_g)

    def gloop(g, carry):
        slot = lax.rem(g, 4)

        @pl.when(g >= 4)
        def _():
            out_dma(g - 4, slot, 0).wait()
            out_dma(g - 4, slot, 1).wait()

        @pl.when(g < _NG)
        def _():
            prep(g, slot)
            pltpu.async_copy(feats_hbm.at[idxbuf.at[slot, 0]],
                             rowsbuf.at[slot, 0], sem_g)
            pltpu.async_copy(feats_hbm.at[idxbuf.at[slot, 1]],
                             rowsbuf.at[slot, 1], sem_g)

        @pl.when(g >= 2)
        def _():
            pslot = lax.rem(g - 2, 4)
            gather_dma(g - 2, pslot, 0).wait()
            gather_dma(g - 2, pslot, 1).wait()
            rb = wid * (_CPT // 2) + (g - 2) * 64
            pltpu.async_copy(
                rowsbuf.at[pslot, 0],
                table_out.at[pl.ds(rb, 64), pl.ds(0, _C)], sem_rows)
            pltpu.async_copy(
                rowsbuf.at[pslot, 1],
                table_out.at[pl.ds(rb, 64), pl.ds(_C, _C)], sem_rows)
        return carry
    lax.fori_loop(0, _NG + 2, gloop, 0)

    for tail in (_NG - 2, _NG - 1):
        out_dma(tail, tail % 4, 0).wait()
        out_dma(tail, tail % 4, 1).wait()


_sc_mesh = plsc.VectorSubcoreMesh(core_axis_name="c", subcore_axis_name="s")
_sc_params = pltpu.CompilerParams(
    needs_layout_passes=False, use_tc_tiling_on_sc=False)

_sc_a_call = pl.kernel(
    _sc_a_body,
    mesh=_sc_mesh,
    compiler_params=pltpu.CompilerParams(
        needs_layout_passes=False, use_tc_tiling_on_sc=True),
    out_type=(
        jax.ShapeDtypeStruct((_M,), jnp.int32),
        jax.ShapeDtypeStruct((_M // 128, 128), jnp.float32),
    ),
    scratch_types=[
        pltpu.VMEM((2 * _CROWS, 128), jnp.int32),
        pltpu.VMEM((_CPT,), jnp.int32),
        pltpu.VMEM((64, 128), jnp.float32),
        pltpu.VMEM_SHARED((_N // 128, 128), jnp.int32),
        pltpu.SemaphoreType.DMA,
        pltpu.SemaphoreType.DMA,
        pltpu.SemaphoreType.DMA,
    ],
)

_sc_b_call = pl.kernel(
    _sc_b_body,
    mesh=_sc_mesh,
    compiler_params=_sc_params,
    out_type=jax.ShapeDtypeStruct((_M // 2, 2 * _C), jnp.float32),
    scratch_types=[
        pltpu.VMEM((_CPT,), jnp.int32),
        pltpu.VMEM((4, 2, 64), jnp.int32),
        pltpu.VMEM((4, 2, 64, _C), jnp.float32),
        pltpu.SemaphoreType.DMA,
        pltpu.SemaphoreType.DMA,
    ],
)


def _tc_prep_body(a_ref, b_ref, o_ref):
    ta = jnp.transpose(a_ref[...])
    tb = jnp.transpose(b_ref[...])
    o_ref[...] = jnp.concatenate([ta, tb], axis=1)


@jax.jit
def _tc_prep(featsT):
    nblk = _N // 2 // 16384
    return pl.pallas_call(
        _tc_prep_body,
        grid=(nblk,),
        in_specs=[
            pl.BlockSpec((_C, 16384), lambda i: (0, i)),
            pl.BlockSpec((_C, 16384), lambda i: (0, i + nblk)),
        ],
        out_specs=pl.BlockSpec((16384, 128), lambda i: (i, 0)),
        out_shape=jax.ShapeDtypeStruct((_N // 2, 128), jnp.float32),
    )(featsT, featsT)


def _tc_body(x_ref, m_ref, o_ref):
    for q in range(8):
        x = x_ref[0, pl.ds(q * 1024, 1024), :]
        t = jnp.transpose(x)
        m = m_ref[pl.ds(q * 16, 16), :].reshape(1, 2048)
        y = jnp.concatenate(
            [t[:_C] * m[:, :1024], t[_C:] * m[:, 1024:]], axis=1)
        for xs in range(8):
            o_ref[0, :, q * 8 + xs, :] = y[:, xs * 256:(xs + 1) * 256]


@jax.jit
def _tc_call(table3, maskm):
    return pl.pallas_call(
        _tc_body,
        grid=(_B, _H // 64),
        in_specs=[
            pl.BlockSpec((1, 8192, 2 * _C), lambda b, h: (b, h, 0)),
            pl.BlockSpec((128, 128), lambda b, h: (b * (_H // 64) + h, 0)),
        ],
        out_specs=pl.BlockSpec((1, _C, 64, _W), lambda b, h: (b, 0, h, 0)),
        out_shape=jax.ShapeDtypeStruct((_B, _C, _H, _W), jnp.float32),
    )(table3, maskm)


def kernel(feats, batch_idx, coords):
    cell = (batch_idx * (_H * _W)
            + coords[:, 0] * _W + coords[:, 1]).astype(jnp.int32)
    feats_lin = _tc_prep(feats.T).reshape(_N, _C)
    winner, maskm = _sc_a_call(cell.reshape(_N // 128, 128))
    table = _sc_b_call(winner, feats_lin)
    return _tc_call(table.reshape(_B, (_H // 8) * 1024, 2 * _C), maskm)
